# Initial kernel scaffold; baseline (speedup 1.0000x reference)
import jax
import jax.numpy as jnp
from jax.experimental import pallas as pl


def kernel(x, edge_index, W_est, b_est, W_gnn, b_gnn, W_cls, b_cls):
    raise NotImplementedError("write your pallas kernel here")


# speedup vs baseline: 32.2566x; 32.2566x over previous
import functools

import jax
import jax.numpy as jnp
from jax import lax
from jax.experimental import pallas as pl
from jax.experimental.pallas import tpu as pltpu
from jax.experimental.pallas import tpu_sc as plsc

N = 10000
E = 160000
F = 256
D = 8

NC = 2
NS = 16
NW = NC * NS
CH = 128
NCHUNK = 40
E_PAD = NW * NCHUNK * CH
N_PAD = 10240
DUMMY = 10016
STRIPE = N_PAD // NS

_mesh = plsc.VectorSubcoreMesh(core_axis_name="c", subcore_axis_name="s",
                               num_cores=NC, num_subcores=NS)
_sc_params = pltpu.CompilerParams(use_tc_tiling_on_sc=False)


def _wid():
    return lax.axis_index("s") * NC + lax.axis_index("c")


@functools.partial(
    pl.kernel,
    out_type=jax.ShapeDtypeStruct((NC, N_PAD, D), jnp.float32),
    mesh=_mesh,
    scratch_types=[
        pltpu.VMEM((NCHUNK, CH), jnp.int32),
        pltpu.VMEM((CH, D), jnp.float32),
        pltpu.VMEM_SHARED((N_PAD, D), jnp.float32),
    ],
    compiler_params=_sc_params,
)
def _deg_kernel(dst_hbm, ones_hbm, zeros_hbm, out_hbm, dst_v, ones_v, acc_sh):
    cid = lax.axis_index("c")
    sid = lax.axis_index("s")
    wid = _wid()
    pltpu.sync_copy(zeros_hbm.at[pl.ds(sid * STRIPE, STRIPE)],
                    acc_sh.at[pl.ds(sid * STRIPE, STRIPE)])
    pltpu.sync_copy(ones_hbm, ones_v)
    pltpu.sync_copy(dst_hbm.at[wid], dst_v)
    plsc.subcore_barrier()

    def body(j, carry):
        pltpu.sync_copy(ones_v, acc_sh.at[dst_v.at[j]], add=True)
        return carry

    lax.fori_loop(0, NCHUNK, body, 0)
    plsc.subcore_barrier()
    pltpu.sync_copy(acc_sh.at[pl.ds(sid * STRIPE, STRIPE)],
                    out_hbm.at[cid, pl.ds(sid * STRIPE, STRIPE)])


@functools.partial(
    pl.kernel,
    out_type=jax.ShapeDtypeStruct((NC, N_PAD, D), jnp.float32),
    mesh=_mesh,
    scratch_types=[
        pltpu.VMEM((NCHUNK, CH), jnp.int32),
        pltpu.VMEM((NCHUNK, CH), jnp.int32),
        pltpu.VMEM((CH, D), jnp.float32),
        pltpu.SemaphoreType.DMA,
        pltpu.VMEM_SHARED((N_PAD, D), jnp.float32),
    ],
    compiler_params=_sc_params,
)
def _edge_kernel(q_hbm, src_hbm, dst_hbm, zeros_hbm, out_hbm,
                 src_v, dst_v, rows_v, sem, acc_sh):
    cid = lax.axis_index("c")
    sid = lax.axis_index("s")
    wid = _wid()
    pltpu.sync_copy(zeros_hbm.at[pl.ds(sid * STRIPE, STRIPE)],
                    acc_sh.at[pl.ds(sid * STRIPE, STRIPE)])
    pltpu.sync_copy(src_hbm.at[wid], src_v)
    pltpu.sync_copy(dst_hbm.at[wid], dst_v)
    plsc.subcore_barrier()

    def body(j, carry):
        pltpu.async_copy(q_hbm.at[src_v.at[j]], rows_v, sem).wait()
        pltpu.sync_copy(rows_v, acc_sh.at[dst_v.at[j]], add=True)
        return carry

    lax.fori_loop(0, NCHUNK, body, 0)
    plsc.subcore_barrier()
    pltpu.sync_copy(acc_sh.at[pl.ds(sid * STRIPE, STRIPE)],
                    out_hbm.at[cid, pl.ds(sid * STRIPE, STRIPE)])


_BN = 1024


def _scale_body(x_ref, wc_ref, d0_ref, d1_ref, q_ref, dinv_ref):
    deg = 1.0 + d0_ref[...] + d1_ref[...]
    dinv = lax.rsqrt(deg)
    p = jnp.dot(x_ref[...], wc_ref[...], preferred_element_type=jnp.float32,
                precision=lax.Precision.HIGHEST)
    q_ref[...] = dinv * p
    dinv_ref[...] = dinv


def _combine_body(a0_ref, a1_ref, q_ref, dinv_ref, bias_ref, out_ref):
    out_ref[...] = (dinv_ref[...] * (a0_ref[...] + a1_ref[...] + q_ref[...])
                    + bias_ref[0:1, :])


def kernel(x, edge_index, W_est, b_est, W_gnn, b_gnn, W_cls, b_cls):
    C = W_cls.shape[1]
    Wc = jnp.concatenate(
        [W_est, W_gnn @ W_cls, jnp.zeros((F, D - 1 - C), jnp.float32)], axis=1)
    bias = jnp.concatenate(
        [b_est, b_gnn @ W_cls + b_cls, jnp.zeros((D - 1 - C,), jnp.float32)])
    bias8 = jnp.broadcast_to(bias[None, :], (8, D))

    pad = jnp.full((E_PAD - E,), DUMMY, jnp.int32)
    src3 = jnp.concatenate([edge_index[0], pad]).reshape(NW, NCHUNK, CH)
    dst3 = jnp.concatenate([edge_index[1], pad]).reshape(NW, NCHUNK, CH)
    ones = jnp.ones((CH, D), jnp.float32)
    zeros = jnp.zeros((N_PAD, D), jnp.float32)

    degp = _deg_kernel(dst3, ones, zeros)

    grid = N_PAD // _BN
    q, dinv8 = pl.pallas_call(
        _scale_body,
        grid=(grid,),
        in_specs=[
            pl.BlockSpec((_BN, F), lambda i: (i, 0)),
            pl.BlockSpec((F, D), lambda i: (0, 0)),
            pl.BlockSpec((_BN, D), lambda i: (i, 0)),
            pl.BlockSpec((_BN, D), lambda i: (i, 0)),
        ],
        out_specs=[
            pl.BlockSpec((_BN, D), lambda i: (i, 0)),
            pl.BlockSpec((_BN, D), lambda i: (i, 0)),
        ],
        out_shape=[
            jax.ShapeDtypeStruct((N_PAD, D), jnp.float32),
            jax.ShapeDtypeStruct((N_PAD, D), jnp.float32),
        ],
    )(x, Wc, degp[0], degp[1])

    accp = _edge_kernel(q, src3, dst3, zeros)

    out8 = pl.pallas_call(
        _combine_body,
        grid=(grid,),
        in_specs=[
            pl.BlockSpec((_BN, D), lambda i: (i, 0)),
            pl.BlockSpec((_BN, D), lambda i: (i, 0)),
            pl.BlockSpec((_BN, D), lambda i: (i, 0)),
            pl.BlockSpec((_BN, D), lambda i: (i, 0)),
            pl.BlockSpec((8, D), lambda i: (0, 0)),
        ],
        out_specs=pl.BlockSpec((_BN, D), lambda i: (i, 0)),
        out_shape=jax.ShapeDtypeStruct((N_PAD, D), jnp.float32),
    )(accp[0], accp[1], q, dinv8, bias8)

    y = out8[:N, 1:1 + C]
    s = out8[:N, 0:1]
    return (y, s)


# speedup vs baseline: 35.9777x; 1.1154x over previous
import functools

import jax
import jax.numpy as jnp
from jax import lax
from jax.experimental import pallas as pl
from jax.experimental.pallas import tpu as pltpu
from jax.experimental.pallas import tpu_sc as plsc

N = 10000
E = 160000
F = 256
D = 8

NC = 2
NS = 16
NW = NC * NS
CH = 128
NCHUNK = 40
E_PAD = NW * NCHUNK * CH
N_PAD = 10240
DUMMY = 10016
STRIPE = N_PAD // NS

_mesh = plsc.VectorSubcoreMesh(core_axis_name="c", subcore_axis_name="s",
                               num_cores=NC, num_subcores=NS)
_sc_params = pltpu.CompilerParams(use_tc_tiling_on_sc=False)


def _wid():
    return lax.axis_index("s") * NC + lax.axis_index("c")


@functools.partial(
    pl.kernel,
    out_type=jax.ShapeDtypeStruct((NC, N_PAD, D), jnp.float32),
    mesh=_mesh,
    scratch_types=[
        pltpu.VMEM((NCHUNK, CH), jnp.int32),
        pltpu.VMEM((CH, D), jnp.float32),
        pltpu.VMEM_SHARED((N_PAD, D), jnp.float32),
    ],
    compiler_params=_sc_params,
)
def _deg_kernel(dst_hbm, ones_hbm, zeros_hbm, out_hbm, dst_v, ones_v, acc_sh):
    cid = lax.axis_index("c")
    sid = lax.axis_index("s")
    wid = _wid()
    pltpu.sync_copy(zeros_hbm.at[pl.ds(sid * STRIPE, STRIPE)],
                    acc_sh.at[pl.ds(sid * STRIPE, STRIPE)])
    pltpu.sync_copy(ones_hbm, ones_v)
    pltpu.sync_copy(dst_hbm.at[wid], dst_v)
    plsc.subcore_barrier()

    def body(j, carry):
        pltpu.sync_copy(ones_v, acc_sh.at[dst_v.at[j]], add=True)
        return carry

    lax.fori_loop(0, NCHUNK, body, 0)
    plsc.subcore_barrier()
    pltpu.sync_copy(acc_sh.at[pl.ds(sid * STRIPE, STRIPE)],
                    out_hbm.at[cid, pl.ds(sid * STRIPE, STRIPE)])


@functools.partial(
    pl.kernel,
    out_type=jax.ShapeDtypeStruct((NC, N_PAD, D), jnp.float32),
    mesh=_mesh,
    scratch_types=[
        pltpu.VMEM((NCHUNK, CH), jnp.int32),
        pltpu.VMEM((NCHUNK, CH), jnp.int32),
        pltpu.VMEM((2, CH, D), jnp.float32),
        pltpu.SemaphoreType.DMA,
        pltpu.SemaphoreType.DMA,
        pltpu.VMEM_SHARED((N_PAD, D), jnp.float32),
    ],
    compiler_params=_sc_params,
)
def _edge_kernel(q_hbm, src_hbm, dst_hbm, zeros_hbm, out_hbm,
                 src_v, dst_v, rows_v, sem0, sem1, acc_sh):
    cid = lax.axis_index("c")
    sid = lax.axis_index("s")
    wid = _wid()
    pltpu.sync_copy(zeros_hbm.at[pl.ds(sid * STRIPE, STRIPE)],
                    acc_sh.at[pl.ds(sid * STRIPE, STRIPE)])
    pltpu.sync_copy(src_hbm.at[wid], src_v)
    pltpu.sync_copy(dst_hbm.at[wid], dst_v)
    plsc.subcore_barrier()

    pltpu.async_copy(q_hbm.at[src_v.at[0]], rows_v.at[0], sem0)

    def step(j, carry):
        pltpu.async_copy(q_hbm.at[src_v.at[2 * j + 1]], rows_v.at[1], sem1)
        pltpu.make_async_copy(q_hbm.at[src_v.at[2 * j]], rows_v.at[0],
                              sem0).wait()
        pltpu.sync_copy(rows_v.at[0], acc_sh.at[dst_v.at[2 * j]], add=True)

        @pl.when(j + 1 < NCHUNK // 2)
        def _():
            pltpu.async_copy(q_hbm.at[src_v.at[2 * j + 2]], rows_v.at[0],
                             sem0)

        pltpu.make_async_copy(q_hbm.at[src_v.at[2 * j + 1]], rows_v.at[1],
                              sem1).wait()
        pltpu.sync_copy(rows_v.at[1], acc_sh.at[dst_v.at[2 * j + 1]],
                        add=True)
        return carry

    lax.fori_loop(0, NCHUNK // 2, step, 0)
    plsc.subcore_barrier()
    pltpu.sync_copy(acc_sh.at[pl.ds(sid * STRIPE, STRIPE)],
                    out_hbm.at[cid, pl.ds(sid * STRIPE, STRIPE)])


_BN = 1024


def _scale_body(x_ref, wc_ref, d0_ref, d1_ref, q_ref, dinv_ref):
    deg = 1.0 + d0_ref[...] + d1_ref[...]
    dinv = lax.rsqrt(deg)
    p = jnp.dot(x_ref[...], wc_ref[...], preferred_element_type=jnp.float32,
                precision=lax.Precision.HIGHEST)
    q_ref[...] = dinv * p
    dinv_ref[...] = dinv


def _combine_body(a0_ref, a1_ref, q_ref, dinv_ref, bias_ref, out_ref):
    out_ref[...] = (dinv_ref[...] * (a0_ref[...] + a1_ref[...] + q_ref[...])
                    + bias_ref[0:1, :])


def kernel(x, edge_index, W_est, b_est, W_gnn, b_gnn, W_cls, b_cls):
    C = W_cls.shape[1]
    Wc = jnp.concatenate(
        [W_est, W_gnn @ W_cls, jnp.zeros((F, D - 1 - C), jnp.float32)], axis=1)
    bias = jnp.concatenate(
        [b_est, b_gnn @ W_cls + b_cls, jnp.zeros((D - 1 - C,), jnp.float32)])
    bias8 = jnp.broadcast_to(bias[None, :], (8, D))

    pad = jnp.full((E_PAD - E,), DUMMY, jnp.int32)
    src3 = jnp.concatenate([edge_index[0], pad]).reshape(NW, NCHUNK, CH)
    dst3 = jnp.concatenate([edge_index[1], pad]).reshape(NW, NCHUNK, CH)
    ones = jnp.ones((CH, D), jnp.float32)
    zeros = jnp.zeros((N_PAD, D), jnp.float32)

    degp = _deg_kernel(dst3, ones, zeros)

    grid = N_PAD // _BN
    q, dinv8 = pl.pallas_call(
        _scale_body,
        grid=(grid,),
        in_specs=[
            pl.BlockSpec((_BN, F), lambda i: (i, 0)),
            pl.BlockSpec((F, D), lambda i: (0, 0)),
            pl.BlockSpec((_BN, D), lambda i: (i, 0)),
            pl.BlockSpec((_BN, D), lambda i: (i, 0)),
        ],
        out_specs=[
            pl.BlockSpec((_BN, D), lambda i: (i, 0)),
            pl.BlockSpec((_BN, D), lambda i: (i, 0)),
        ],
        out_shape=[
            jax.ShapeDtypeStruct((N_PAD, D), jnp.float32),
            jax.ShapeDtypeStruct((N_PAD, D), jnp.float32),
        ],
    )(x, Wc, degp[0], degp[1])

    accp = _edge_kernel(q, src3, dst3, zeros)

    out8 = pl.pallas_call(
        _combine_body,
        grid=(grid,),
        in_specs=[
            pl.BlockSpec((_BN, D), lambda i: (i, 0)),
            pl.BlockSpec((_BN, D), lambda i: (i, 0)),
            pl.BlockSpec((_BN, D), lambda i: (i, 0)),
            pl.BlockSpec((_BN, D), lambda i: (i, 0)),
            pl.BlockSpec((8, D), lambda i: (0, 0)),
        ],
        out_specs=pl.BlockSpec((_BN, D), lambda i: (i, 0)),
        out_shape=jax.ShapeDtypeStruct((N_PAD, D), jnp.float32),
    )(accp[0], accp[1], q, dinv8, bias8)

    y = out8[:N, 1:1 + C]
    s = out8[:N, 0:1]
    return (y, s)


# speedup vs baseline: 44.2698x; 1.2305x over previous
import functools

import jax
import jax.numpy as jnp
from jax import lax
from jax.experimental import pallas as pl
from jax.experimental.pallas import tpu as pltpu
from jax.experimental.pallas import tpu_sc as plsc

N = 10000
E = 160000
F = 256
D = 8

NC = 2
NS = 16
NW = NC * NS
CH = 128
NCHUNK = 40
NBUF = 8
LOOK = 4
E_PAD = NW * NCHUNK * CH
N_PAD = 10240
DUMMY = 10016
STRIPE = N_PAD // NS
NR = N_PAD * D // 128

_mesh = plsc.VectorSubcoreMesh(core_axis_name="c", subcore_axis_name="s",
                               num_cores=NC, num_subcores=NS)
_sc_params = pltpu.CompilerParams(use_tc_tiling_on_sc=False)


def _wid():
    return lax.axis_index("s") * NC + lax.axis_index("c")


@functools.partial(
    pl.kernel,
    out_type=jax.ShapeDtypeStruct((NC, N_PAD, D), jnp.float32),
    mesh=_mesh,
    scratch_types=[
        pltpu.VMEM((NCHUNK, CH), jnp.int32),
        pltpu.VMEM((CH, D), jnp.float32),
        pltpu.SemaphoreType.DMA,
        pltpu.VMEM_SHARED((N_PAD, D), jnp.float32),
    ],
    compiler_params=_sc_params,
)
def _deg_kernel(dst_hbm, ones_hbm, zeros_hbm, out_hbm, dst_v, ones_v, sem,
                acc_sh):
    cid = lax.axis_index("c")
    sid = lax.axis_index("s")
    wid = _wid()
    pltpu.sync_copy(zeros_hbm.at[pl.ds(sid * STRIPE, STRIPE)],
                    acc_sh.at[pl.ds(sid * STRIPE, STRIPE)])
    pltpu.sync_copy(ones_hbm, ones_v)
    pltpu.sync_copy(dst_hbm.at[wid], dst_v)
    plsc.subcore_barrier()

    pltpu.async_copy(ones_v, acc_sh.at[dst_v.at[0]], sem, add=True)
    pltpu.async_copy(ones_v, acc_sh.at[dst_v.at[1]], sem, add=True)

    def body(j, carry):
        pltpu.make_async_copy(ones_v, acc_sh.at[dst_v.at[0]], sem).wait()
        pltpu.async_copy(ones_v, acc_sh.at[dst_v.at[j + 2]], sem, add=True)
        return carry

    lax.fori_loop(0, NCHUNK - 2, body, 0)
    pltpu.make_async_copy(ones_v, acc_sh.at[dst_v.at[0]], sem).wait()
    pltpu.make_async_copy(ones_v, acc_sh.at[dst_v.at[0]], sem).wait()
    plsc.subcore_barrier()
    pltpu.sync_copy(acc_sh.at[pl.ds(sid * STRIPE, STRIPE)],
                    out_hbm.at[cid, pl.ds(sid * STRIPE, STRIPE)])


@functools.partial(
    pl.kernel,
    out_type=jax.ShapeDtypeStruct((NC, N_PAD, D), jnp.float32),
    mesh=_mesh,
    scratch_types=[
        pltpu.VMEM((NCHUNK, CH), jnp.int32),
        pltpu.VMEM((NCHUNK, CH), jnp.int32),
        pltpu.VMEM((NBUF, CH, D), jnp.float32),
        [pltpu.SemaphoreType.DMA] * NBUF,
        [pltpu.SemaphoreType.DMA] * NBUF,
        pltpu.VMEM_SHARED((N_PAD, D), jnp.float32),
    ],
    compiler_params=_sc_params,
)
def _edge_kernel(q_hbm, src_hbm, dst_hbm, zeros_hbm, out_hbm,
                 src_v, dst_v, rows_v, gsems, ssems, acc_sh):
    cid = lax.axis_index("c")
    sid = lax.axis_index("s")
    wid = _wid()
    pltpu.sync_copy(zeros_hbm.at[pl.ds(sid * STRIPE, STRIPE)],
                    acc_sh.at[pl.ds(sid * STRIPE, STRIPE)])
    pltpu.sync_copy(src_hbm.at[wid], src_v)
    pltpu.sync_copy(dst_hbm.at[wid], dst_v)
    plsc.subcore_barrier()

    def _gwait(b):
        pltpu.make_async_copy(q_hbm.at[src_v.at[0]], rows_v.at[b],
                              gsems[b]).wait()

    def _swait(b):
        pltpu.make_async_copy(rows_v.at[b], acc_sh.at[dst_v.at[0]],
                              ssems[b]).wait()

    for b in range(LOOK):
        pltpu.async_copy(q_hbm.at[src_v.at[b]], rows_v.at[b], gsems[b])

    for j in range(NBUF - LOOK):
        pltpu.async_copy(q_hbm.at[src_v.at[j + LOOK]],
                         rows_v.at[(j + LOOK) % NBUF], gsems[(j + LOOK) % NBUF])
        _gwait(j % NBUF)
        pltpu.async_copy(rows_v.at[j % NBUF], acc_sh.at[dst_v.at[j]],
                         ssems[j % NBUF], add=True)

    def outer(g, carry):
        for b in range(NBUF):
            j = (NBUF - LOOK) + g * NBUF + b
            sf = b
            _swait(sf)
            pltpu.async_copy(q_hbm.at[src_v.at[j + LOOK]], rows_v.at[sf],
                             gsems[sf])
            _gwait((b + NBUF - LOOK) % NBUF)
            pltpu.async_copy(rows_v.at[(b + NBUF - LOOK) % NBUF],
                             acc_sh.at[dst_v.at[j]],
                             ssems[(b + NBUF - LOOK) % NBUF], add=True)
        return carry

    lax.fori_loop(0, (NCHUNK - NBUF) // NBUF, outer, 0)
    for j in range(NCHUNK - LOOK, NCHUNK):
        _gwait(j % NBUF)
        pltpu.async_copy(rows_v.at[j % NBUF], acc_sh.at[dst_v.at[j]],
                         ssems[j % NBUF], add=True)
    for b in range(NBUF):
        _swait(b)
    plsc.subcore_barrier()
    pltpu.sync_copy(acc_sh.at[pl.ds(sid * STRIPE, STRIPE)],
                    out_hbm.at[cid, pl.ds(sid * STRIPE, STRIPE)])


NR0 = N_PAD // 16
_BR = 128
_NGRID = NR0 // _BR


def _scale_body(x3a_ref, x3b_ref, wg_ref, we_ref, wy_ref, d0_ref, d1_ref,
                q_ref, dinv_ref):
    deg = 1.0 + d0_ref[...] + d1_ref[...]
    dinv = lax.rsqrt(deg)
    wc8 = we_ref[...] + jnp.dot(wg_ref[...], wy_ref[...],
                                preferred_element_type=jnp.float32)
    ps = [jnp.dot(x3a_ref[u], wc8, preferred_element_type=jnp.float32)
          for u in range(15)]
    ps.append(jnp.dot(x3b_ref[...], wc8, preferred_element_type=jnp.float32))
    q_ref[...] = dinv * jnp.concatenate(ps, axis=1)
    dinv_ref[...] = dinv


def _combine_body(a0_ref, a1_ref, q_ref, dinv_ref, bias_ref, out_ref):
    out_ref[...] = (dinv_ref[...] * (a0_ref[...] + a1_ref[...] + q_ref[...])
                    + bias_ref[0:1, :])


def kernel(x, edge_index, W_est, b_est, W_gnn, b_gnn, W_cls, b_cls):
    C = W_cls.shape[1]
    H = W_gnn.shape[1]
    we = jnp.concatenate([W_est, jnp.zeros((F, D - 1), jnp.float32)], axis=1)
    wy = jnp.concatenate(
        [jnp.zeros((H, 1), jnp.float32), W_cls,
         jnp.zeros((H, D - 1 - C), jnp.float32)], axis=1)
    bias = jnp.concatenate(
        [b_est, b_gnn @ W_cls + b_cls, jnp.zeros((D - 1 - C,), jnp.float32)])
    bias128 = jnp.broadcast_to(jnp.tile(bias, 128 // D)[None, :], (8, 128))

    pad = jnp.full((E_PAD - E,), DUMMY, jnp.int32)
    srcp = jnp.concatenate([edge_index[0], pad])
    dstp = jnp.concatenate([edge_index[1], pad])
    srcp = 16 * (srcp % NR0) + srcp // NR0
    dstp = 16 * (dstp % NR0) + dstp // NR0
    src3 = srcp.reshape(NW, NCHUNK, CH)
    dst3 = dstp.reshape(NW, NCHUNK, CH)
    ones = jnp.ones((CH, D), jnp.float32)
    zeros = jnp.zeros((N_PAD, D), jnp.float32)

    x3a = x[:15 * NR0].reshape(15, NR0, F)
    x3b = jnp.concatenate(
        [x[15 * NR0:], jnp.zeros((N_PAD - N, F), jnp.float32)], axis=0)

    degp = _deg_kernel(dst3, ones, zeros)
    degp_r = degp.reshape(NC, NR, 128)

    grid = _NGRID
    q_r, dinv_r = pl.pallas_call(
        _scale_body,
        grid=(grid,),
        in_specs=[
            pl.BlockSpec((15, _BR, F), lambda i: (0, i, 0)),
            pl.BlockSpec((_BR, F), lambda i: (i, 0)),
            pl.BlockSpec((F, H), lambda i: (0, 0)),
            pl.BlockSpec((F, D), lambda i: (0, 0)),
            pl.BlockSpec((H, D), lambda i: (0, 0)),
            pl.BlockSpec((_BR, 128), lambda i: (i, 0)),
            pl.BlockSpec((_BR, 128), lambda i: (i, 0)),
        ],
        out_specs=[
            pl.BlockSpec((_BR, 128), lambda i: (i, 0)),
            pl.BlockSpec((_BR, 128), lambda i: (i, 0)),
        ],
        out_shape=[
            jax.ShapeDtypeStruct((NR, 128), jnp.float32),
            jax.ShapeDtypeStruct((NR, 128), jnp.float32),
        ],
    )(x3a, x3b, W_gnn, we, wy, degp_r[0], degp_r[1])

    accp = _edge_kernel(q_r.reshape(N_PAD, D), src3, dst3, zeros)
    accp_r = accp.reshape(NC, NR, 128)

    out8 = pl.pallas_call(
        _combine_body,
        grid=(grid,),
        in_specs=[
            pl.BlockSpec((_BR, 128), lambda i: (i, 0)),
            pl.BlockSpec((_BR, 128), lambda i: (i, 0)),
            pl.BlockSpec((_BR, 128), lambda i: (i, 0)),
            pl.BlockSpec((_BR, 128), lambda i: (i, 0)),
            pl.BlockSpec((8, 128), lambda i: (0, 0)),
        ],
        out_specs=pl.BlockSpec((_BR, 128), lambda i: (i, 0)),
        out_shape=jax.ShapeDtypeStruct((NR, 128), jnp.float32),
    )(accp_r[0], accp_r[1], q_r, dinv_r, bias128)

    outn = out8.reshape(NR0, 16, D).transpose(1, 0, 2).reshape(N_PAD, D)
    y = outn[:N, 1:1 + C]
    s = outn[:N, 0:1]
    return (y, s)


# speedup vs baseline: 60.4125x; 1.3646x over previous
import functools

import jax
import jax.numpy as jnp
from jax import lax
from jax.experimental import pallas as pl
from jax.experimental.pallas import tpu as pltpu
from jax.experimental.pallas import tpu_sc as plsc

N = 10000
E = 160000
F = 256
D = 8

NC = 2
NS = 16
NW = NC * NS
CH = 128
NCHUNK = 40
NBUF = 8
LOOK = 4
E_PAD = NW * NCHUNK * CH
N_PAD = 10240
DUMMY = 10016
STRIPE = N_PAD // NS
NR = N_PAD * D // 128

_mesh = plsc.VectorSubcoreMesh(core_axis_name="c", subcore_axis_name="s",
                               num_cores=NC, num_subcores=NS)
_sc_params = pltpu.CompilerParams(use_tc_tiling_on_sc=False)


def _wid():
    return lax.axis_index("s") * NC + lax.axis_index("c")


STR_V = STRIPE * D // 128


@functools.partial(
    pl.kernel,
    out_type=[jax.ShapeDtypeStruct((N_PAD, D), jnp.float32),
              jax.ShapeDtypeStruct((N_PAD, D), jnp.float32)],
    mesh=_mesh,
    scratch_types=[
        pltpu.VMEM((NCHUNK, CH), jnp.int32),
        pltpu.VMEM((CH, D), jnp.float32),
        pltpu.SemaphoreType.DMA,
        pltpu.VMEM_SHARED((N_PAD, D), jnp.float32),
    ],
    compiler_params=_sc_params,
)
def _deg_kernel(dst_hbm, ones_hbm, zeros_hbm, out0_hbm, out1_hbm, dst_v,
                ones_v, sem, acc_sh):
    cid = lax.axis_index("c")
    sid = lax.axis_index("s")
    wid = _wid()
    pltpu.sync_copy(zeros_hbm.at[pl.ds(sid * STRIPE, STRIPE)],
                    acc_sh.at[pl.ds(sid * STRIPE, STRIPE)])
    pltpu.sync_copy(ones_hbm, ones_v)
    pltpu.sync_copy(dst_hbm.at[wid], dst_v)
    plsc.subcore_barrier()

    pltpu.async_copy(ones_v, acc_sh.at[dst_v.at[0]], sem, add=True)
    pltpu.async_copy(ones_v, acc_sh.at[dst_v.at[1]], sem, add=True)

    def body(j, carry):
        pltpu.make_async_copy(ones_v, acc_sh.at[dst_v.at[0]], sem).wait()
        pltpu.async_copy(ones_v, acc_sh.at[dst_v.at[j + 2]], sem, add=True)
        return carry

    lax.fori_loop(0, NCHUNK - 2, body, 0)
    pltpu.make_async_copy(ones_v, acc_sh.at[dst_v.at[0]], sem).wait()
    pltpu.make_async_copy(ones_v, acc_sh.at[dst_v.at[0]], sem).wait()
    plsc.subcore_barrier()

    @pl.when(cid == 0)
    def _():
        pltpu.sync_copy(acc_sh.at[pl.ds(sid * STRIPE, STRIPE)],
                        out0_hbm.at[pl.ds(sid * STRIPE, STRIPE)])

    @pl.when(cid == 1)
    def _():
        pltpu.sync_copy(acc_sh.at[pl.ds(sid * STRIPE, STRIPE)],
                        out1_hbm.at[pl.ds(sid * STRIPE, STRIPE)])


@functools.partial(
    pl.kernel,
    out_type=[jax.ShapeDtypeStruct((N_PAD, D), jnp.float32),
              jax.ShapeDtypeStruct((N_PAD, D), jnp.float32)],
    mesh=_mesh,
    scratch_types=[
        pltpu.VMEM((NCHUNK, CH), jnp.int32),
        pltpu.VMEM((NCHUNK, CH), jnp.int32),
        pltpu.VMEM((NBUF, CH, D), jnp.float32),
        [pltpu.SemaphoreType.DMA] * NBUF,
        [pltpu.SemaphoreType.DMA] * NBUF,
        pltpu.VMEM_SHARED((N_PAD, D), jnp.float32),
    ],
    compiler_params=_sc_params,
)
def _edge_kernel(q_hbm, src_hbm, dst_hbm, zeros_hbm, out0_hbm, out1_hbm,
                 src_v, dst_v, rows_v, gsems, ssems, acc_sh):
    cid = lax.axis_index("c")
    sid = lax.axis_index("s")
    wid = _wid()
    pltpu.sync_copy(zeros_hbm.at[pl.ds(sid * STRIPE, STRIPE)],
                    acc_sh.at[pl.ds(sid * STRIPE, STRIPE)])
    pltpu.sync_copy(src_hbm.at[wid], src_v)
    pltpu.sync_copy(dst_hbm.at[wid], dst_v)
    plsc.subcore_barrier()

    def _gwait(b):
        pltpu.make_async_copy(q_hbm.at[src_v.at[0]], rows_v.at[b],
                              gsems[b]).wait()

    def _swait(b):
        pltpu.make_async_copy(rows_v.at[b], acc_sh.at[dst_v.at[0]],
                              ssems[b]).wait()

    for b in range(LOOK):
        pltpu.async_copy(q_hbm.at[src_v.at[b]], rows_v.at[b], gsems[b])

    for j in range(NBUF - LOOK):
        pltpu.async_copy(q_hbm.at[src_v.at[j + LOOK]],
                         rows_v.at[(j + LOOK) % NBUF], gsems[(j + LOOK) % NBUF])
        _gwait(j % NBUF)
        pltpu.async_copy(rows_v.at[j % NBUF], acc_sh.at[dst_v.at[j]],
                         ssems[j % NBUF], add=True)

    def outer(g, carry):
        for b in range(NBUF):
            j = (NBUF - LOOK) + g * NBUF + b
            sf = b
            _swait(sf)
            pltpu.async_copy(q_hbm.at[src_v.at[j + LOOK]], rows_v.at[sf],
                             gsems[sf])
            _gwait((b + NBUF - LOOK) % NBUF)
            pltpu.async_copy(rows_v.at[(b + NBUF - LOOK) % NBUF],
                             acc_sh.at[dst_v.at[j]],
                             ssems[(b + NBUF - LOOK) % NBUF], add=True)
        return carry

    lax.fori_loop(0, (NCHUNK - NBUF) // NBUF, outer, 0)
    for j in range(NCHUNK - LOOK, NCHUNK):
        _gwait(j % NBUF)
        pltpu.async_copy(rows_v.at[j % NBUF], acc_sh.at[dst_v.at[j]],
                         ssems[j % NBUF], add=True)
    for b in range(NBUF):
        _swait(b)
    plsc.subcore_barrier()

    @pl.when(cid == 0)
    def _():
        pltpu.sync_copy(acc_sh.at[pl.ds(sid * STRIPE, STRIPE)],
                        out0_hbm.at[pl.ds(sid * STRIPE, STRIPE)])

    @pl.when(cid == 1)
    def _():
        pltpu.sync_copy(acc_sh.at[pl.ds(sid * STRIPE, STRIPE)],
                        out1_hbm.at[pl.ds(sid * STRIPE, STRIPE)])


NR0 = N_PAD // 16
_BR = 128
_NGRID = NR0 // _BR


def _scale_body(x3a_ref, x3b_ref, wg_ref, we_ref, wy_ref, d0_ref, d1_ref,
                q_ref, dinv_ref):
    deg = 1.0 + d0_ref[...] + d1_ref[...]
    dinv = lax.rsqrt(deg)
    wc8 = we_ref[...] + jnp.dot(wg_ref[...], wy_ref[...],
                                preferred_element_type=jnp.float32)
    ps = [jnp.dot(x3a_ref[u], wc8, preferred_element_type=jnp.float32)
          for u in range(15)]
    ps.append(jnp.dot(x3b_ref[...], wc8, preferred_element_type=jnp.float32))
    q_ref[...] = dinv * jnp.concatenate(ps, axis=1)
    dinv_ref[...] = dinv


def _combine_body(a0_ref, a1_ref, q_ref, dinv_ref, b0_ref, bg_ref, wy_ref,
                  out_ref):
    bias8 = b0_ref[0:1, :] + jnp.dot(bg_ref[...], wy_ref[...],
                                     preferred_element_type=jnp.float32)[0:1, :]
    bias128 = jnp.concatenate([bias8] * (128 // D), axis=1)
    out_ref[...] = (dinv_ref[...] * (a0_ref[...] + a1_ref[...] + q_ref[...])
                    + bias128)


def kernel(x, edge_index, W_est, b_est, W_gnn, b_gnn, W_cls, b_cls):
    C = W_cls.shape[1]
    H = W_gnn.shape[1]
    we = jnp.concatenate([W_est, jnp.zeros((F, D - 1), jnp.float32)], axis=1)
    wy = jnp.concatenate(
        [jnp.zeros((H, 1), jnp.float32), W_cls,
         jnp.zeros((H, D - 1 - C), jnp.float32)], axis=1)
    bias0 = jnp.concatenate(
        [b_est, b_cls, jnp.zeros((D - 1 - C,), jnp.float32)])
    bias0_8 = jnp.broadcast_to(bias0[None, :], (8, D))
    bgnn8 = jnp.broadcast_to(b_gnn[None, :], (8, H))

    pad = jnp.full((E_PAD - E,), DUMMY, jnp.int32)
    srcp = jnp.concatenate([edge_index[0], pad])
    dstp = jnp.concatenate([edge_index[1], pad])
    srcp = 16 * (srcp % NR0) + srcp // NR0
    dstp = 16 * (dstp % NR0) + dstp // NR0
    src3 = srcp.reshape(NW, NCHUNK, CH)
    dst3 = dstp.reshape(NW, NCHUNK, CH)
    ones = jnp.ones((CH, D), jnp.float32)
    zeros = jnp.zeros((N_PAD, D), jnp.float32)

    x3a = x[:15 * NR0].reshape(15, NR0, F)
    x3b = jnp.concatenate(
        [x[15 * NR0:], jnp.zeros((N_PAD - N, F), jnp.float32)], axis=0)

    deg0, deg1 = _deg_kernel(dst3, ones, zeros)
    deg0_r = deg0.reshape(NR, 128)
    deg1_r = deg1.reshape(NR, 128)

    grid = _NGRID
    q_r, dinv_r = pl.pallas_call(
        _scale_body,
        grid=(grid,),
        in_specs=[
            pl.BlockSpec((15, _BR, F), lambda i: (0, i, 0)),
            pl.BlockSpec((_BR, F), lambda i: (i, 0)),
            pl.BlockSpec((F, H), lambda i: (0, 0)),
            pl.BlockSpec((F, D), lambda i: (0, 0)),
            pl.BlockSpec((H, D), lambda i: (0, 0)),
            pl.BlockSpec((_BR, 128), lambda i: (i, 0)),
            pl.BlockSpec((_BR, 128), lambda i: (i, 0)),
        ],
        out_specs=[
            pl.BlockSpec((_BR, 128), lambda i: (i, 0)),
            pl.BlockSpec((_BR, 128), lambda i: (i, 0)),
        ],
        out_shape=[
            jax.ShapeDtypeStruct((NR, 128), jnp.float32),
            jax.ShapeDtypeStruct((NR, 128), jnp.float32),
        ],
    )(x3a, x3b, W_gnn, we, wy, deg0_r, deg1_r)

    acc0, acc1 = _edge_kernel(q_r.reshape(N_PAD, D), src3, dst3, zeros)
    acc0_r = acc0.reshape(NR, 128)
    acc1_r = acc1.reshape(NR, 128)

    out8 = pl.pallas_call(
        _combine_body,
        grid=(grid,),
        in_specs=[
            pl.BlockSpec((_BR, 128), lambda i: (i, 0)),
            pl.BlockSpec((_BR, 128), lambda i: (i, 0)),
            pl.BlockSpec((_BR, 128), lambda i: (i, 0)),
            pl.BlockSpec((_BR, 128), lambda i: (i, 0)),
            pl.BlockSpec((8, D), lambda i: (0, 0)),
            pl.BlockSpec((8, H), lambda i: (0, 0)),
            pl.BlockSpec((H, D), lambda i: (0, 0)),
        ],
        out_specs=pl.BlockSpec((_BR, 128), lambda i: (i, 0)),
        out_shape=jax.ShapeDtypeStruct((NR, 128), jnp.float32),
    )(acc0_r, acc1_r, q_r, dinv_r, bias0_8, bgnn8, wy)

    outn = out8.reshape(NR0, 16, D).transpose(1, 0, 2).reshape(N_PAD, D)
    y = outn[:N, 1:1 + C]
    s = outn[:N, 0:1]
    return (y, s)


# speedup vs baseline: 65.1812x; 1.0789x over previous
import functools

import jax
import jax.numpy as jnp
from jax import lax
from jax.experimental import pallas as pl
from jax.experimental.pallas import tpu as pltpu
from jax.experimental.pallas import tpu_sc as plsc

N = 10000
E = 160000
F = 256
D = 8

NC = 2
NS = 16
NW = NC * NS
CH = 128
NCHUNK = 40
NBUF = 8
LOOK = 4
E_PAD = NW * NCHUNK * CH
N_PAD = 10240
DUMMY = 10016
STRIPE = N_PAD // NS
NR = N_PAD * D // 128

_mesh = plsc.VectorSubcoreMesh(core_axis_name="c", subcore_axis_name="s",
                               num_cores=NC, num_subcores=NS)
_sc_params = pltpu.CompilerParams(use_tc_tiling_on_sc=False)


def _wid():
    return lax.axis_index("s") * NC + lax.axis_index("c")


STR_V = STRIPE * D // 128


@functools.partial(
    pl.kernel,
    out_type=[jax.ShapeDtypeStruct((N_PAD, D), jnp.float32),
              jax.ShapeDtypeStruct((N_PAD, D), jnp.float32)],
    mesh=_mesh,
    scratch_types=[
        pltpu.VMEM((NCHUNK, CH), jnp.int32),
        pltpu.VMEM((CH, D), jnp.float32),
        pltpu.SemaphoreType.DMA,
        pltpu.VMEM_SHARED((N_PAD, D), jnp.float32),
    ],
    compiler_params=_sc_params,
)
def _deg_kernel(dst_hbm, ones_hbm, zeros_hbm, out0_hbm, out1_hbm, dst_v,
                ones_v, sem, acc_sh):
    cid = lax.axis_index("c")
    sid = lax.axis_index("s")
    wid = _wid()
    pltpu.sync_copy(zeros_hbm.at[pl.ds(sid * STRIPE, STRIPE)],
                    acc_sh.at[pl.ds(sid * STRIPE, STRIPE)])
    pltpu.sync_copy(ones_hbm, ones_v)
    pltpu.sync_copy(dst_hbm.at[wid], dst_v)
    plsc.subcore_barrier()

    pltpu.async_copy(ones_v, acc_sh.at[dst_v.at[0]], sem, add=True)
    pltpu.async_copy(ones_v, acc_sh.at[dst_v.at[1]], sem, add=True)

    def body(j, carry):
        pltpu.make_async_copy(ones_v, acc_sh.at[dst_v.at[0]], sem).wait()
        pltpu.async_copy(ones_v, acc_sh.at[dst_v.at[j + 2]], sem, add=True)
        return carry

    lax.fori_loop(0, NCHUNK - 2, body, 0)
    pltpu.make_async_copy(ones_v, acc_sh.at[dst_v.at[0]], sem).wait()
    pltpu.make_async_copy(ones_v, acc_sh.at[dst_v.at[0]], sem).wait()
    plsc.subcore_barrier()

    @pl.when(cid == 0)
    def _():
        pltpu.sync_copy(acc_sh.at[pl.ds(sid * STRIPE, STRIPE)],
                        out0_hbm.at[pl.ds(sid * STRIPE, STRIPE)])

    @pl.when(cid == 1)
    def _():
        pltpu.sync_copy(acc_sh.at[pl.ds(sid * STRIPE, STRIPE)],
                        out1_hbm.at[pl.ds(sid * STRIPE, STRIPE)])


@functools.partial(
    pl.kernel,
    out_type=[jax.ShapeDtypeStruct((N_PAD, D), jnp.float32),
              jax.ShapeDtypeStruct((N_PAD, D), jnp.float32)],
    mesh=_mesh,
    scratch_types=[
        pltpu.VMEM((NCHUNK, CH), jnp.int32),
        pltpu.VMEM((NCHUNK, CH), jnp.int32),
        pltpu.VMEM((NBUF, CH, D), jnp.float32),
        [pltpu.SemaphoreType.DMA] * NBUF,
        [pltpu.SemaphoreType.DMA] * NBUF,
        pltpu.VMEM_SHARED((N_PAD, D), jnp.float32),
    ],
    compiler_params=_sc_params,
)
def _edge_kernel(q_hbm, src_hbm, dst_hbm, zeros_hbm, out0_hbm, out1_hbm,
                 src_v, dst_v, rows_v, gsems, ssems, acc_sh):
    cid = lax.axis_index("c")
    sid = lax.axis_index("s")
    wid = _wid()
    pltpu.sync_copy(zeros_hbm.at[pl.ds(sid * STRIPE, STRIPE)],
                    acc_sh.at[pl.ds(sid * STRIPE, STRIPE)])
    pltpu.sync_copy(src_hbm.at[wid], src_v)
    pltpu.sync_copy(dst_hbm.at[wid], dst_v)
    plsc.subcore_barrier()

    def _gwait(b):
        pltpu.make_async_copy(q_hbm.at[src_v.at[0]], rows_v.at[b],
                              gsems[b]).wait()

    def _swait(b):
        pltpu.make_async_copy(rows_v.at[b], acc_sh.at[dst_v.at[0]],
                              ssems[b]).wait()

    for b in range(LOOK):
        pltpu.async_copy(q_hbm.at[src_v.at[b]], rows_v.at[b], gsems[b])

    for j in range(NBUF - LOOK):
        pltpu.async_copy(q_hbm.at[src_v.at[j + LOOK]],
                         rows_v.at[(j + LOOK) % NBUF], gsems[(j + LOOK) % NBUF])
        _gwait(j % NBUF)
        pltpu.async_copy(rows_v.at[j % NBUF], acc_sh.at[dst_v.at[j]],
                         ssems[j % NBUF], add=True)

    def outer(g, carry):
        for b in range(NBUF):
            j = (NBUF - LOOK) + g * NBUF + b
            sf = b
            _swait(sf)
            pltpu.async_copy(q_hbm.at[src_v.at[j + LOOK]], rows_v.at[sf],
                             gsems[sf])
            _gwait((b + NBUF - LOOK) % NBUF)
            pltpu.async_copy(rows_v.at[(b + NBUF - LOOK) % NBUF],
                             acc_sh.at[dst_v.at[j]],
                             ssems[(b + NBUF - LOOK) % NBUF], add=True)
        return carry

    lax.fori_loop(0, (NCHUNK - NBUF) // NBUF, outer, 0)
    for j in range(NCHUNK - LOOK, NCHUNK):
        _gwait(j % NBUF)
        pltpu.async_copy(rows_v.at[j % NBUF], acc_sh.at[dst_v.at[j]],
                         ssems[j % NBUF], add=True)
    for b in range(NBUF):
        _swait(b)
    plsc.subcore_barrier()

    @pl.when(cid == 0)
    def _():
        pltpu.sync_copy(acc_sh.at[pl.ds(sid * STRIPE, STRIPE)],
                        out0_hbm.at[pl.ds(sid * STRIPE, STRIPE)])

    @pl.when(cid == 1)
    def _():
        pltpu.sync_copy(acc_sh.at[pl.ds(sid * STRIPE, STRIPE)],
                        out1_hbm.at[pl.ds(sid * STRIPE, STRIPE)])


NR0 = N_PAD // 16
_BR = 128
_NGRID = NR0 // _BR


def _scale_body(*refs):
    (x3b_ref, wg_ref, we_ref, wy_ref, d0_ref, d1_ref) = refs[15:21]
    q_ref, dinv_ref = refs[21], refs[22]
    deg = 1.0 + d0_ref[...] + d1_ref[...]
    dinv = lax.rsqrt(deg)
    wc8 = we_ref[...] + jnp.dot(wg_ref[...], wy_ref[...],
                                preferred_element_type=jnp.float32)
    ps = [jnp.dot(refs[u][...], wc8, preferred_element_type=jnp.float32)
          for u in range(15)]
    ps.append(jnp.dot(x3b_ref[...], wc8, preferred_element_type=jnp.float32))
    q_ref[...] = dinv * jnp.concatenate(ps, axis=1)
    dinv_ref[...] = dinv


def _combine_body(a0_ref, a1_ref, q_ref, dinv_ref, b0_ref, bg_ref, wy_ref,
                  out_ref):
    bias8 = b0_ref[0:1, :] + jnp.dot(bg_ref[...], wy_ref[...],
                                     preferred_element_type=jnp.float32)[0:1, :]
    bias128 = jnp.concatenate([bias8] * (128 // D), axis=1)
    out_ref[...] = (dinv_ref[...] * (a0_ref[...] + a1_ref[...] + q_ref[...])
                    + bias128)


def kernel(x, edge_index, W_est, b_est, W_gnn, b_gnn, W_cls, b_cls):
    C = W_cls.shape[1]
    H = W_gnn.shape[1]
    we = jnp.concatenate([W_est, jnp.zeros((F, D - 1), jnp.float32)], axis=1)
    wy = jnp.concatenate(
        [jnp.zeros((H, 1), jnp.float32), W_cls,
         jnp.zeros((H, D - 1 - C), jnp.float32)], axis=1)
    bias0 = jnp.concatenate(
        [b_est, b_cls, jnp.zeros((D - 1 - C,), jnp.float32)])
    bias0_8 = jnp.broadcast_to(bias0[None, :], (8, D))
    bgnn8 = jnp.broadcast_to(b_gnn[None, :], (8, H))

    pad2 = jnp.full((2, E_PAD - E), DUMMY, jnp.int32)
    ei = jnp.concatenate([edge_index, pad2], axis=1)
    ei = 16 * (ei % NR0) + ei // NR0
    ei3 = ei.reshape(2, NW, NCHUNK, CH)
    src3 = ei3[0]
    dst3 = ei3[1]
    ones = jnp.ones((CH, D), jnp.float32)
    zeros = jnp.zeros((N_PAD, D), jnp.float32)

    x3b = jnp.concatenate(
        [x[15 * NR0:], jnp.zeros((N_PAD - N, F), jnp.float32)], axis=0)

    deg0, deg1 = _deg_kernel(dst3, ones, zeros)
    deg0_r = deg0.reshape(NR, 128)
    deg1_r = deg1.reshape(NR, 128)

    grid = _NGRID
    q_r, dinv_r = pl.pallas_call(
        _scale_body,
        grid=(grid,),
        in_specs=(
            [pl.BlockSpec((_BR, F), lambda i, u=u: (5 * u + i, 0))
             for u in range(15)] +
            [
                pl.BlockSpec((_BR, F), lambda i: (i, 0)),
                pl.BlockSpec((F, H), lambda i: (0, 0)),
                pl.BlockSpec((F, D), lambda i: (0, 0)),
                pl.BlockSpec((H, D), lambda i: (0, 0)),
                pl.BlockSpec((_BR, 128), lambda i: (i, 0)),
                pl.BlockSpec((_BR, 128), lambda i: (i, 0)),
            ]
        ),
        out_specs=[
            pl.BlockSpec((_BR, 128), lambda i: (i, 0)),
            pl.BlockSpec((_BR, 128), lambda i: (i, 0)),
        ],
        out_shape=[
            jax.ShapeDtypeStruct((NR, 128), jnp.float32),
            jax.ShapeDtypeStruct((NR, 128), jnp.float32),
        ],
    )(*([x] * 15), x3b, W_gnn, we, wy, deg0_r, deg1_r)

    acc0, acc1 = _edge_kernel(q_r.reshape(N_PAD, D), src3, dst3, zeros)
    acc0_r = acc0.reshape(NR, 128)
    acc1_r = acc1.reshape(NR, 128)

    out8 = pl.pallas_call(
        _combine_body,
        grid=(grid,),
        in_specs=[
            pl.BlockSpec((_BR, 128), lambda i: (i, 0)),
            pl.BlockSpec((_BR, 128), lambda i: (i, 0)),
            pl.BlockSpec((_BR, 128), lambda i: (i, 0)),
            pl.BlockSpec((_BR, 128), lambda i: (i, 0)),
            pl.BlockSpec((8, D), lambda i: (0, 0)),
            pl.BlockSpec((8, H), lambda i: (0, 0)),
            pl.BlockSpec((H, D), lambda i: (0, 0)),
        ],
        out_specs=pl.BlockSpec((_BR, 128), lambda i: (i, 0)),
        out_shape=jax.ShapeDtypeStruct((NR, 128), jnp.float32),
    )(acc0_r, acc1_r, q_r, dinv_r, bias0_8, bgnn8, wy)

    outn = out8.reshape(NR0, 16, D).transpose(1, 0, 2).reshape(N_PAD, D)
    y = outn[:N, 1:1 + C]
    s = outn[:N, 0:1]
    return (y, s)


# speedup vs baseline: 80.7047x; 1.2382x over previous
import functools

import jax
import jax.numpy as jnp
from jax import lax
from jax.experimental import pallas as pl
from jax.experimental.pallas import tpu as pltpu
from jax.experimental.pallas import tpu_sc as plsc

N = 10000
E = 160000
F = 256
D = 8

NC = 2
NS = 16
NW = NC * NS
CH = 128
NCHUNK = 40
NBUF = 8
LOOK = 4
E_PAD = NW * NCHUNK * CH
N_PAD = 10240
DUMMY = 10016
STRIPE = N_PAD // NS
NR = N_PAD * D // 128

_mesh = plsc.VectorSubcoreMesh(core_axis_name="c", subcore_axis_name="s",
                               num_cores=NC, num_subcores=NS)
_sc_params = pltpu.CompilerParams(use_tc_tiling_on_sc=False)


def _wid():
    return lax.axis_index("s") * NC + lax.axis_index("c")


STR_V = STRIPE * D // 128


@functools.partial(
    pl.kernel,
    out_type=[jax.ShapeDtypeStruct((N_PAD, D), jnp.float32),
              jax.ShapeDtypeStruct((N_PAD, D), jnp.float32)],
    mesh=_mesh,
    scratch_types=[
        pltpu.VMEM((NCHUNK, CH), jnp.int32),
        pltpu.VMEM((CH, D), jnp.float32),
        pltpu.SemaphoreType.DMA,
        pltpu.VMEM_SHARED((N_PAD, D), jnp.float32),
    ],
    compiler_params=_sc_params,
)
def _deg_kernel(dst_hbm, ones_hbm, zeros_hbm, out0_hbm, out1_hbm, dst_v,
                ones_v, sem, acc_sh):
    cid = lax.axis_index("c")
    sid = lax.axis_index("s")
    wid = _wid()
    pltpu.sync_copy(zeros_hbm.at[pl.ds(sid * STRIPE, STRIPE)],
                    acc_sh.at[pl.ds(sid * STRIPE, STRIPE)])
    pltpu.sync_copy(ones_hbm, ones_v)
    pltpu.sync_copy(dst_hbm.at[wid], dst_v)
    plsc.subcore_barrier()

    pltpu.async_copy(ones_v, acc_sh.at[dst_v.at[0]], sem, add=True)
    pltpu.async_copy(ones_v, acc_sh.at[dst_v.at[1]], sem, add=True)

    def body(j, carry):
        pltpu.make_async_copy(ones_v, acc_sh.at[dst_v.at[0]], sem).wait()
        pltpu.async_copy(ones_v, acc_sh.at[dst_v.at[j + 2]], sem, add=True)
        return carry

    lax.fori_loop(0, NCHUNK - 2, body, 0)
    pltpu.make_async_copy(ones_v, acc_sh.at[dst_v.at[0]], sem).wait()
    pltpu.make_async_copy(ones_v, acc_sh.at[dst_v.at[0]], sem).wait()
    plsc.subcore_barrier()

    @pl.when(cid == 0)
    def _():
        pltpu.sync_copy(acc_sh.at[pl.ds(sid * STRIPE, STRIPE)],
                        out0_hbm.at[pl.ds(sid * STRIPE, STRIPE)])

    @pl.when(cid == 1)
    def _():
        pltpu.sync_copy(acc_sh.at[pl.ds(sid * STRIPE, STRIPE)],
                        out1_hbm.at[pl.ds(sid * STRIPE, STRIPE)])


@functools.partial(
    pl.kernel,
    out_type=[jax.ShapeDtypeStruct((N_PAD, D), jnp.float32),
              jax.ShapeDtypeStruct((N_PAD, D), jnp.float32)],
    mesh=_mesh,
    scratch_types=[
        pltpu.VMEM((NCHUNK, CH), jnp.int32),
        pltpu.VMEM((NCHUNK, CH), jnp.int32),
        pltpu.VMEM((NBUF, CH, D), jnp.float32),
        [pltpu.SemaphoreType.DMA] * NBUF,
        [pltpu.SemaphoreType.DMA] * NBUF,
        pltpu.VMEM_SHARED((N_PAD, D), jnp.float32),
        pltpu.VMEM_SHARED((N_PAD, D), jnp.float32),
    ],
    compiler_params=_sc_params,
)
def _edge_kernel(q_hbm, src_hbm, dst_hbm, zeros_hbm, out0_hbm, out1_hbm,
                 src_v, dst_v, rows_v, gsems, ssems, acc_sh, q_sh):
    cid = lax.axis_index("c")
    sid = lax.axis_index("s")
    wid = _wid()
    pltpu.sync_copy(zeros_hbm.at[pl.ds(sid * STRIPE, STRIPE)],
                    acc_sh.at[pl.ds(sid * STRIPE, STRIPE)])
    pltpu.sync_copy(q_hbm.at[pl.ds(sid * STRIPE, STRIPE)],
                    q_sh.at[pl.ds(sid * STRIPE, STRIPE)])
    pltpu.sync_copy(src_hbm.at[wid], src_v)
    pltpu.sync_copy(dst_hbm.at[wid], dst_v)
    plsc.subcore_barrier()

    def _gwait(b):
        pltpu.make_async_copy(q_sh.at[src_v.at[0]], rows_v.at[b],
                              gsems[b]).wait()

    def _swait(b):
        pltpu.make_async_copy(rows_v.at[b], acc_sh.at[dst_v.at[0]],
                              ssems[b]).wait()

    for b in range(LOOK):
        pltpu.async_copy(q_sh.at[src_v.at[b]], rows_v.at[b], gsems[b])

    for j in range(NBUF - LOOK):
        pltpu.async_copy(q_sh.at[src_v.at[j + LOOK]],
                         rows_v.at[(j + LOOK) % NBUF], gsems[(j + LOOK) % NBUF])
        _gwait(j % NBUF)
        pltpu.async_copy(rows_v.at[j % NBUF], acc_sh.at[dst_v.at[j]],
                         ssems[j % NBUF], add=True)

    def outer(g, carry):
        for b in range(NBUF):
            j = (NBUF - LOOK) + g * NBUF + b
            sf = b
            _swait(sf)
            pltpu.async_copy(q_sh.at[src_v.at[j + LOOK]], rows_v.at[sf],
                             gsems[sf])
            _gwait((b + NBUF - LOOK) % NBUF)
            pltpu.async_copy(rows_v.at[(b + NBUF - LOOK) % NBUF],
                             acc_sh.at[dst_v.at[j]],
                             ssems[(b + NBUF - LOOK) % NBUF], add=True)
        return carry

    lax.fori_loop(0, (NCHUNK - NBUF) // NBUF, outer, 0)
    for j in range(NCHUNK - LOOK, NCHUNK):
        _gwait(j % NBUF)
        pltpu.async_copy(rows_v.at[j % NBUF], acc_sh.at[dst_v.at[j]],
                         ssems[j % NBUF], add=True)
    for b in range(NBUF):
        _swait(b)
    plsc.subcore_barrier()

    @pl.when(cid == 0)
    def _():
        pltpu.sync_copy(acc_sh.at[pl.ds(sid * STRIPE, STRIPE)],
                        out0_hbm.at[pl.ds(sid * STRIPE, STRIPE)])

    @pl.when(cid == 1)
    def _():
        pltpu.sync_copy(acc_sh.at[pl.ds(sid * STRIPE, STRIPE)],
                        out1_hbm.at[pl.ds(sid * STRIPE, STRIPE)])


NR0 = N_PAD // 16
_BR = 128
_NGRID = NR0 // _BR


def _scale_body(*refs):
    (x3b_ref, wg_ref, we_ref, wy_ref, d0_ref, d1_ref) = refs[15:21]
    q_ref, dinv_ref = refs[21], refs[22]
    deg = 1.0 + d0_ref[...] + d1_ref[...]
    dinv = lax.rsqrt(deg)
    wc8 = we_ref[...] + jnp.dot(wg_ref[...], wy_ref[...],
                                preferred_element_type=jnp.float32)
    ps = [jnp.dot(refs[u][...], wc8, preferred_element_type=jnp.float32)
          for u in range(15)]
    ps.append(jnp.dot(x3b_ref[...], wc8, preferred_element_type=jnp.float32))
    q_ref[...] = dinv * jnp.concatenate(ps, axis=1)
    dinv_ref[...] = dinv


def _combine_body(a0_ref, a1_ref, q_ref, dinv_ref, b0_ref, bg_ref, wy_ref,
                  out_ref):
    bias8 = b0_ref[0:1, :] + jnp.dot(bg_ref[...], wy_ref[...],
                                     preferred_element_type=jnp.float32)[0:1, :]
    bias128 = jnp.concatenate([bias8] * (128 // D), axis=1)
    out_ref[...] = (dinv_ref[...] * (a0_ref[...] + a1_ref[...] + q_ref[...])
                    + bias128)


def kernel(x, edge_index, W_est, b_est, W_gnn, b_gnn, W_cls, b_cls):
    C = W_cls.shape[1]
    H = W_gnn.shape[1]
    we = jnp.concatenate([W_est, jnp.zeros((F, D - 1), jnp.float32)], axis=1)
    wy = jnp.concatenate(
        [jnp.zeros((H, 1), jnp.float32), W_cls,
         jnp.zeros((H, D - 1 - C), jnp.float32)], axis=1)
    bias0 = jnp.concatenate(
        [b_est, b_cls, jnp.zeros((D - 1 - C,), jnp.float32)])
    bias0_8 = jnp.broadcast_to(bias0[None, :], (8, D))
    bgnn8 = jnp.broadcast_to(b_gnn[None, :], (8, H))

    pad2 = jnp.full((2, E_PAD - E), DUMMY, jnp.int32)
    ei = jnp.concatenate([edge_index, pad2], axis=1)
    ei = 16 * (ei % NR0) + ei // NR0
    ei3 = ei.reshape(2, NW, NCHUNK, CH)
    src3 = ei3[0]
    dst3 = ei3[1]
    ones = jnp.ones((CH, D), jnp.float32)
    zeros = jnp.zeros((N_PAD, D), jnp.float32)

    x3b = jnp.concatenate(
        [x[15 * NR0:], jnp.zeros((N_PAD - N, F), jnp.float32)], axis=0)

    deg0, deg1 = _deg_kernel(dst3, ones, zeros)
    deg0_r = deg0.reshape(NR, 128)
    deg1_r = deg1.reshape(NR, 128)

    grid = _NGRID
    q_r, dinv_r = pl.pallas_call(
        _scale_body,
        grid=(grid,),
        in_specs=(
            [pl.BlockSpec((_BR, F), lambda i, u=u: (5 * u + i, 0))
             for u in range(15)] +
            [
                pl.BlockSpec((_BR, F), lambda i: (i, 0)),
                pl.BlockSpec((F, H), lambda i: (0, 0)),
                pl.BlockSpec((F, D), lambda i: (0, 0)),
                pl.BlockSpec((H, D), lambda i: (0, 0)),
                pl.BlockSpec((_BR, 128), lambda i: (i, 0)),
                pl.BlockSpec((_BR, 128), lambda i: (i, 0)),
            ]
        ),
        out_specs=[
            pl.BlockSpec((_BR, 128), lambda i: (i, 0)),
            pl.BlockSpec((_BR, 128), lambda i: (i, 0)),
        ],
        out_shape=[
            jax.ShapeDtypeStruct((NR, 128), jnp.float32),
            jax.ShapeDtypeStruct((NR, 128), jnp.float32),
        ],
    )(*([x] * 15), x3b, W_gnn, we, wy, deg0_r, deg1_r)

    acc0, acc1 = _edge_kernel(q_r.reshape(N_PAD, D), src3, dst3, zeros)
    acc0_r = acc0.reshape(NR, 128)
    acc1_r = acc1.reshape(NR, 128)

    out8 = pl.pallas_call(
        _combine_body,
        grid=(grid,),
        in_specs=[
            pl.BlockSpec((_BR, 128), lambda i: (i, 0)),
            pl.BlockSpec((_BR, 128), lambda i: (i, 0)),
            pl.BlockSpec((_BR, 128), lambda i: (i, 0)),
            pl.BlockSpec((_BR, 128), lambda i: (i, 0)),
            pl.BlockSpec((8, D), lambda i: (0, 0)),
            pl.BlockSpec((8, H), lambda i: (0, 0)),
            pl.BlockSpec((H, D), lambda i: (0, 0)),
        ],
        out_specs=pl.BlockSpec((_BR, 128), lambda i: (i, 0)),
        out_shape=jax.ShapeDtypeStruct((NR, 128), jnp.float32),
    )(acc0_r, acc1_r, q_r, dinv_r, bias0_8, bgnn8, wy)

    outn = out8.reshape(NR0, 16, D).transpose(1, 0, 2).reshape(N_PAD, D)
    y = outn[:N, 1:1 + C]
    s = outn[:N, 0:1]
    return (y, s)
